# Initial kernel scaffold; baseline (speedup 1.0000x reference)
import functools

import jax
import jax.numpy as jnp
from jax import lax
from jax.experimental import pallas as pl
from jax.experimental.pallas import tpu as pltpu
from jax.experimental.pallas import tpu_sc as plsc

N_NODES = 10000
D_FEAT = 128
D_HID = 128
D_OUT = 2

NC = 2
NS = 16
NW = NC * NS
N_PAD = 10240
RPT = N_PAD // NS
EC = 128
CH = 80
E_PAD = NW * CH * EC
DPAD = 16

BR = 256
GRID = N_PAD // BR

_MESH = plsc.VectorSubcoreMesh(core_axis_name="c", subcore_axis_name="s")


@functools.partial(
    pl.kernel,
    out_type=jax.ShapeDtypeStruct((NC, N_PAD, DPAD), jnp.float32),
    mesh=_MESH,
    scratch_types=[
        pltpu.VMEM((CH, EC), jnp.int32),
        pltpu.VMEM((EC, DPAD), jnp.float32),
        pltpu.VMEM_SHARED((N_PAD, DPAD), jnp.float32),
    ],
)
def _sc_degree(dst_hbm, ones_hbm, zero_hbm, out_hbm, idst, ones_v, acc):
    c = lax.axis_index("c")
    s = lax.axis_index("s")
    base = s * RPT
    pltpu.sync_copy(zero_hbm, acc.at[pl.ds(base, RPT)])
    pltpu.sync_copy(ones_hbm, ones_v)
    pltpu.sync_copy(dst_hbm.at[c, s], idst)
    plsc.subcore_barrier()

    def body(j, carry):
        pltpu.sync_copy(ones_v, acc.at[idst.at[j]], add=True)
        return carry

    lax.fori_loop(0, CH, body, 0)
    plsc.subcore_barrier()
    pltpu.sync_copy(acc.at[pl.ds(base, RPT)], out_hbm.at[c, pl.ds(base, RPT)])


def _make_sc_segsum(D):
    @functools.partial(
        pl.kernel,
        out_type=jax.ShapeDtypeStruct((NC, N_PAD, D), jnp.float32),
        mesh=_MESH,
        scratch_types=[
            pltpu.VMEM((CH, EC), jnp.int32),
            pltpu.VMEM((CH, EC), jnp.int32),
            pltpu.VMEM((EC, D), jnp.float32),
            pltpu.VMEM_SHARED((N_PAD, D), jnp.float32),
            pltpu.SemaphoreType.DMA,
        ],
    )
    def seg(table_hbm, src_hbm, dst_hbm, zero_hbm, out_hbm,
            isrc, idst, rows, acc, sem):
        c = lax.axis_index("c")
        s = lax.axis_index("s")
        base = s * RPT
        pltpu.sync_copy(zero_hbm, acc.at[pl.ds(base, RPT)])
        pltpu.sync_copy(src_hbm.at[c, s], isrc)
        pltpu.sync_copy(dst_hbm.at[c, s], idst)
        plsc.subcore_barrier()

        def body(j, carry):
            pltpu.async_copy(table_hbm.at[isrc.at[j]], rows, sem).wait()
            pltpu.sync_copy(rows, acc.at[idst.at[j]], add=True)
            return carry

        lax.fori_loop(0, CH, body, 0)
        plsc.subcore_barrier()
        pltpu.sync_copy(acc.at[pl.ds(base, RPT)],
                        out_hbm.at[c, pl.ds(base, RPT)])

    return seg


_sc_segsum_hid = _make_sc_segsum(D_HID)
_sc_segsum_out = _make_sc_segsum(DPAD)


def _dinv_block(degp_blk):
    dsum = degp_blk[0] + degp_blk[1]
    return lax.rsqrt(dsum[:, 0:1] + 1.0)


def _tc_hs(x_ref, w_ref, degp_ref, hs_ref):
    dinv = _dinv_block(degp_ref)
    hs_ref[...] = jnp.dot(x_ref[...], w_ref[...],
                          preferred_element_type=jnp.float32) * dinv


def _tc_layer2(accp_ref, hs_ref, degp_ref, w2_ref, b1_ref, gs_ref):
    dinv = _dinv_block(degp_ref)
    out1 = dinv * (accp_ref[0] + accp_ref[1] + hs_ref[...]) + b1_ref[...]
    r = jnp.maximum(out1, 0.0)
    gs_ref[...] = jnp.dot(r, w2_ref[...],
                          preferred_element_type=jnp.float32) * dinv


def _tc_softmax(acc2p_ref, gs_ref, degp_ref, b2_ref, out_ref):
    dinv = _dinv_block(degp_ref)
    t = dinv * (acc2p_ref[0] + acc2p_ref[1] + gs_ref[...]) + b2_ref[...]
    l0 = t[:, 0:1]
    l1 = t[:, 1:2]
    m = jnp.maximum(l0, l1)
    e0 = jnp.exp(l0 - m)
    e1 = jnp.exp(l1 - m)
    inv = 1.0 / (e0 + e1)
    out_ref[...] = jnp.concatenate([e0 * inv, e1 * inv], axis=1)


def kernel(x, edge_index, W1, b1, W2, b2):
    ei = edge_index.astype(jnp.int32)
    src = jnp.concatenate(
        [ei[0], jnp.zeros((E_PAD - ei.shape[1],), jnp.int32)])
    dst = jnp.concatenate(
        [ei[1], jnp.full((E_PAD - ei.shape[1],), N_NODES, jnp.int32)])
    src_r = src.reshape(NC, NS, CH, EC)
    dst_r = dst.reshape(NC, NS, CH, EC)

    x_pad = jnp.zeros((N_PAD, D_FEAT), jnp.float32).at[:N_NODES].set(x)
    w2_pad = jnp.zeros((D_HID, DPAD), jnp.float32).at[:, :D_OUT].set(W2)
    b1r = b1.reshape(1, D_HID)
    b2r = jnp.zeros((1, DPAD), jnp.float32).at[0, :D_OUT].set(b2)
    zero16 = jnp.zeros((RPT, DPAD), jnp.float32)
    zero128 = jnp.zeros((RPT, D_HID), jnp.float32)
    ones16 = jnp.ones((EC, DPAD), jnp.float32)

    degp = _sc_degree(dst_r, ones16, zero16)

    hs = pl.pallas_call(
        _tc_hs,
        grid=(GRID,),
        in_specs=[
            pl.BlockSpec((BR, D_FEAT), lambda i: (i, 0)),
            pl.BlockSpec((D_FEAT, D_HID), lambda i: (0, 0)),
            pl.BlockSpec((NC, BR, DPAD), lambda i: (0, i, 0)),
        ],
        out_specs=pl.BlockSpec((BR, D_HID), lambda i: (i, 0)),
        out_shape=jax.ShapeDtypeStruct((N_PAD, D_HID), jnp.float32),
    )(x_pad, W1, degp)

    accp = _sc_segsum_hid(hs, src_r, dst_r, zero128)

    gs = pl.pallas_call(
        _tc_layer2,
        grid=(GRID,),
        in_specs=[
            pl.BlockSpec((NC, BR, D_HID), lambda i: (0, i, 0)),
            pl.BlockSpec((BR, D_HID), lambda i: (i, 0)),
            pl.BlockSpec((NC, BR, DPAD), lambda i: (0, i, 0)),
            pl.BlockSpec((D_HID, DPAD), lambda i: (0, 0)),
            pl.BlockSpec((1, D_HID), lambda i: (0, 0)),
        ],
        out_specs=pl.BlockSpec((BR, DPAD), lambda i: (i, 0)),
        out_shape=jax.ShapeDtypeStruct((N_PAD, DPAD), jnp.float32),
    )(accp, hs, degp, w2_pad, b1r)

    acc2p = _sc_segsum_out(gs, src_r, dst_r, zero16)

    out = pl.pallas_call(
        _tc_softmax,
        grid=(GRID,),
        in_specs=[
            pl.BlockSpec((NC, BR, DPAD), lambda i: (0, i, 0)),
            pl.BlockSpec((BR, DPAD), lambda i: (i, 0)),
            pl.BlockSpec((NC, BR, DPAD), lambda i: (0, i, 0)),
            pl.BlockSpec((1, DPAD), lambda i: (0, 0)),
        ],
        out_specs=pl.BlockSpec((BR, D_OUT), lambda i: (i, 0)),
        out_shape=jax.ShapeDtypeStruct((N_PAD, D_OUT), jnp.float32),
    )(acc2p, gs, degp, b2r)

    return out[:N_NODES]


# speedup vs baseline: 12.3284x; 12.3284x over previous
import functools

import jax
import jax.numpy as jnp
from jax import lax
from jax.experimental import pallas as pl
from jax.experimental.pallas import tpu as pltpu
from jax.experimental.pallas import tpu_sc as plsc

N_NODES = 10000
D_FEAT = 128
D_HID = 128
D_OUT = 2

NC = 2
NS = 16
NW = NC * NS
N_PAD = 10240
RPT = N_PAD // NS
EC = 128
CH = 80
EPW = CH * EC
E_PAD = NW * EPW
NV = EPW // 16

BR = 256
GRID = N_PAD // BR

_MESH = plsc.VectorSubcoreMesh(core_axis_name="c", subcore_axis_name="s")
_NOLAYOUT = pltpu.CompilerParams(needs_layout_passes=False)


@functools.partial(
    pl.kernel,
    out_type=jax.ShapeDtypeStruct((NC, NS, N_PAD), jnp.float32),
    mesh=_MESH,
    compiler_params=_NOLAYOUT,
    scratch_types=[
        pltpu.VMEM((CH, EC), jnp.int32),
        pltpu.VMEM((N_PAD,), jnp.float32),
    ],
)
def _sc_degree(dst_hbm, zero_hbm, out_hbm, idx_v, acc):
    c = lax.axis_index("c")
    s = lax.axis_index("s")
    pltpu.sync_copy(zero_hbm, acc)
    pltpu.sync_copy(dst_hbm.at[c, s], idx_v)
    ones = jnp.ones((16,), jnp.float32)

    def body(j, carry):
        for k in range(EC // 16):
            ix = idx_v[j, pl.ds(k * 16, 16)]
            plsc.addupdate_scatter(acc, [ix], ones)
        return carry

    lax.fori_loop(0, CH, body, 0)
    pltpu.sync_copy(acc, out_hbm.at[c, s])


@functools.partial(
    pl.kernel,
    out_type=jax.ShapeDtypeStruct((NC, N_PAD, D_HID), jnp.float32),
    mesh=_MESH,
    scratch_types=[
        pltpu.VMEM((CH, EC), jnp.int32),
        pltpu.VMEM((CH, EC), jnp.int32),
        pltpu.VMEM((EC, D_HID), jnp.float32),
        pltpu.VMEM_SHARED((N_PAD, D_HID), jnp.float32),
        pltpu.SemaphoreType.DMA,
    ],
)
def _sc_segsum_hid(table_hbm, src_hbm, dst_hbm, zero_hbm, out_hbm,
                   isrc, idst, rows, acc, sem):
    c = lax.axis_index("c")
    s = lax.axis_index("s")
    base = s * RPT
    pltpu.sync_copy(zero_hbm, acc.at[pl.ds(base, RPT)])
    pltpu.sync_copy(src_hbm.at[c, s], isrc)
    pltpu.sync_copy(dst_hbm.at[c, s], idst)
    plsc.subcore_barrier()

    def body(j, carry):
        pltpu.async_copy(table_hbm.at[isrc.at[j]], rows, sem).wait()
        pltpu.async_copy(rows, acc.at[idst.at[j]], sem, add=True).wait()
        return carry

    lax.fori_loop(0, CH, body, 0)
    plsc.subcore_barrier()
    pltpu.sync_copy(acc.at[pl.ds(base, RPT)],
                    out_hbm.at[c, pl.ds(base, RPT)])


@functools.partial(
    pl.kernel,
    out_type=jax.ShapeDtypeStruct((NC, NS, 2 * N_PAD), jnp.float32),
    mesh=_MESH,
    compiler_params=_NOLAYOUT,
    scratch_types=[
        pltpu.VMEM((CH, EC), jnp.int32),
        pltpu.VMEM((CH, EC), jnp.int32),
        pltpu.VMEM((2 * N_PAD,), jnp.float32),
        pltpu.VMEM((2 * N_PAD,), jnp.float32),
    ],
)
def _sc_segsum_out(table_hbm, src_hbm, dst_hbm, zero_hbm, out_hbm,
                   isrc, idst, table_v, acc):
    c = lax.axis_index("c")
    s = lax.axis_index("s")
    pltpu.sync_copy(table_hbm, table_v)
    pltpu.sync_copy(zero_hbm, acc)
    pltpu.sync_copy(src_hbm.at[c, s], isrc)
    pltpu.sync_copy(dst_hbm.at[c, s], idst)

    def body(j, carry):
        for k in range(EC // 16):
            sb = isrc[j, pl.ds(k * 16, 16)] * 2
            db = idst[j, pl.ds(k * 16, 16)] * 2
            g0 = plsc.load_gather(table_v, [sb])
            g1 = plsc.load_gather(table_v, [sb + 1])
            plsc.addupdate_scatter(acc, [db], g0)
            plsc.addupdate_scatter(acc, [db + 1], g1)
        return carry

    lax.fori_loop(0, CH, body, 0)
    pltpu.sync_copy(acc, out_hbm.at[c, s])


def _dinv_block(deg_blk):
    deg = jnp.sum(deg_blk, axis=1, keepdims=True) + 1.0
    return lax.rsqrt(deg)


def _tc_hs(x_ref, w_ref, deg_ref, hs_ref):
    dinv = _dinv_block(deg_ref[...])
    hs_ref[...] = jnp.dot(x_ref[...], w_ref[...],
                          preferred_element_type=jnp.float32) * dinv


def _tc_layer2(accp_ref, hs_ref, deg_ref, w2_ref, b1_ref, gs_ref):
    dinv = _dinv_block(deg_ref[...])
    out1 = dinv * (accp_ref[0] + accp_ref[1] + hs_ref[...]) + b1_ref[...]
    r = jnp.maximum(out1, 0.0)
    gs_ref[...] = jnp.dot(r, w2_ref[...],
                          preferred_element_type=jnp.float32) * dinv


def _tc_softmax(acc2_ref, gs_ref, deg_ref, b2_ref, out_ref):
    dinv = _dinv_block(deg_ref[...])
    acc2 = jnp.sum(acc2_ref[...], axis=2)
    t = dinv * (acc2 + gs_ref[...]) + b2_ref[...]
    l0 = t[:, 0:1]
    l1 = t[:, 1:2]
    m = jnp.maximum(l0, l1)
    e0 = jnp.exp(l0 - m)
    e1 = jnp.exp(l1 - m)
    inv = 1.0 / (e0 + e1)
    out_ref[...] = jnp.concatenate([e0 * inv, e1 * inv], axis=1)


def kernel(x, edge_index, W1, b1, W2, b2):
    ei = edge_index.astype(jnp.int32)
    src = jnp.concatenate(
        [ei[0], jnp.zeros((E_PAD - ei.shape[1],), jnp.int32)])
    dst = jnp.concatenate(
        [ei[1], jnp.full((E_PAD - ei.shape[1],), N_NODES, jnp.int32)])
    src128 = src.reshape(NC, NS, CH, EC)
    dst128 = dst.reshape(NC, NS, CH, EC)

    x_pad = jnp.zeros((N_PAD, D_FEAT), jnp.float32).at[:N_NODES].set(x)
    b1r = b1.reshape(1, D_HID)
    b2r = b2.reshape(1, D_OUT)
    zero1 = jnp.zeros((N_PAD,), jnp.float32)
    zero2 = jnp.zeros((2 * N_PAD,), jnp.float32)
    zero128 = jnp.zeros((RPT, D_HID), jnp.float32)

    degp = _sc_degree(dst128, zero1)
    deg_t = degp.reshape(NW, N_PAD).transpose(1, 0)

    hs = pl.pallas_call(
        _tc_hs,
        grid=(GRID,),
        in_specs=[
            pl.BlockSpec((BR, D_FEAT), lambda i: (i, 0)),
            pl.BlockSpec((D_FEAT, D_HID), lambda i: (0, 0)),
            pl.BlockSpec((BR, NW), lambda i: (i, 0)),
        ],
        out_specs=pl.BlockSpec((BR, D_HID), lambda i: (i, 0)),
        out_shape=jax.ShapeDtypeStruct((N_PAD, D_HID), jnp.float32),
    )(x_pad, W1, deg_t)

    accp = _sc_segsum_hid(hs, src128, dst128, zero128)

    gs = pl.pallas_call(
        _tc_layer2,
        grid=(GRID,),
        in_specs=[
            pl.BlockSpec((NC, BR, D_HID), lambda i: (0, i, 0)),
            pl.BlockSpec((BR, D_HID), lambda i: (i, 0)),
            pl.BlockSpec((BR, NW), lambda i: (i, 0)),
            pl.BlockSpec((D_HID, D_OUT), lambda i: (0, 0)),
            pl.BlockSpec((1, D_HID), lambda i: (0, 0)),
        ],
        out_specs=pl.BlockSpec((BR, D_OUT), lambda i: (i, 0)),
        out_shape=jax.ShapeDtypeStruct((N_PAD, D_OUT), jnp.float32),
    )(accp, hs, deg_t, W2, b1r)

    acc2p = _sc_segsum_out(gs.reshape(2 * N_PAD), src128, dst128, zero2)
    acc2_t = acc2p.reshape(NW, N_PAD, 2).transpose(1, 2, 0)

    out = pl.pallas_call(
        _tc_softmax,
        grid=(GRID,),
        in_specs=[
            pl.BlockSpec((BR, D_OUT, NW), lambda i: (i, 0, 0)),
            pl.BlockSpec((BR, D_OUT), lambda i: (i, 0)),
            pl.BlockSpec((BR, NW), lambda i: (i, 0)),
            pl.BlockSpec((1, D_OUT), lambda i: (0, 0)),
        ],
        out_specs=pl.BlockSpec((BR, D_OUT), lambda i: (i, 0)),
        out_shape=jax.ShapeDtypeStruct((N_PAD, D_OUT), jnp.float32),
    )(acc2_t, gs, deg_t, b2r)

    return out[:N_NODES]


# speedup vs baseline: 13.0415x; 1.0578x over previous
import functools

import jax
import jax.numpy as jnp
from jax import lax
from jax.experimental import pallas as pl
from jax.experimental.pallas import tpu as pltpu
from jax.experimental.pallas import tpu_sc as plsc

N_NODES = 10000
D_FEAT = 128
D_HID = 128
D_OUT = 2

NC = 2
NS = 16
NW = NC * NS
N_PAD = 10240
RPT = N_PAD // NS
EC = 128
CH = 80
EPW = CH * EC
E_PAD = NW * EPW
NV = EPW // 16

BR = 256
GRID = N_PAD // BR

_MESH = plsc.VectorSubcoreMesh(core_axis_name="c", subcore_axis_name="s")
_NOLAYOUT = pltpu.CompilerParams(needs_layout_passes=False)


@functools.partial(
    pl.kernel,
    out_type=jax.ShapeDtypeStruct((NC, NS, N_PAD), jnp.float32),
    mesh=_MESH,
    compiler_params=_NOLAYOUT,
    scratch_types=[
        pltpu.VMEM((CH, EC), jnp.int32),
        pltpu.VMEM((N_PAD,), jnp.float32),
    ],
)
def _sc_degree(dst_hbm, zero_hbm, out_hbm, idx_v, acc):
    c = lax.axis_index("c")
    s = lax.axis_index("s")
    pltpu.sync_copy(zero_hbm, acc)
    pltpu.sync_copy(dst_hbm.at[c, s], idx_v)
    ones = jnp.ones((16,), jnp.float32)

    def body(j, carry):
        for k in range(EC // 16):
            ix = idx_v[j, pl.ds(k * 16, 16)]
            plsc.addupdate_scatter(acc, [ix], ones)
        return carry

    lax.fori_loop(0, CH, body, 0)
    pltpu.sync_copy(acc, out_hbm.at[c, s])


@functools.partial(
    pl.kernel,
    out_type=jax.ShapeDtypeStruct((NC, N_PAD, D_HID), jnp.float32),
    mesh=_MESH,
    scratch_types=[
        pltpu.VMEM((CH // 5, EC), jnp.int32),
        pltpu.VMEM((CH // 5, EC), jnp.int32),
        pltpu.VMEM((2, EC, D_HID), jnp.float32),
        pltpu.VMEM_SHARED((N_PAD, D_HID), jnp.float32),
        pltpu.SemaphoreType.DMA,
        pltpu.SemaphoreType.DMA,
    ],
)
def _sc_segsum_hid(table_hbm, src_hbm, dst_hbm, zero_hbm, out_hbm,
                   isrc, idst, rows, acc, s0, s1):
    c = lax.axis_index("c")
    s = lax.axis_index("s")
    sems = (s0, s1)
    base = s * RPT
    QC = CH // 5
    pltpu.sync_copy(zero_hbm, acc.at[pl.ds(base, RPT)])
    plsc.subcore_barrier()

    def g_start(j, b):
        pltpu.async_copy(table_hbm.at[isrc.at[j]], rows.at[b], sems[b])

    def g_wait(j, b):
        pltpu.make_async_copy(table_hbm.at[isrc.at[j]], rows.at[b],
                              sems[b]).wait()

    def scat(j, b):
        pltpu.async_copy(rows.at[b], acc.at[idst.at[j]], sems[b],
                         add=True).wait()

    for q in range(5):
        pltpu.sync_copy(src_hbm.at[c, s, pl.ds(q * QC, QC)], isrc)
        pltpu.sync_copy(dst_hbm.at[c, s, pl.ds(q * QC, QC)], idst)
        g_start(0, 0)
        g_start(1, 1)

        def group(g, carry):
            for b in range(2):
                j = 2 * g + b
                g_wait(j, b)
                scat(j, b)
                g_start(j + 2, b)
            return carry

        lax.fori_loop(0, QC // 2 - 1, group, 0)
        for b in range(2):
            j = QC - 2 + b
            g_wait(j, b)
            scat(j, b)
    plsc.subcore_barrier()
    pltpu.sync_copy(acc.at[pl.ds(base, RPT)],
                    out_hbm.at[c, pl.ds(base, RPT)])


@functools.partial(
    pl.kernel,
    out_type=jax.ShapeDtypeStruct((NC, NS, 2 * N_PAD), jnp.float32),
    mesh=_MESH,
    compiler_params=_NOLAYOUT,
    scratch_types=[
        pltpu.VMEM((CH, EC), jnp.int32),
        pltpu.VMEM((CH, EC), jnp.int32),
        pltpu.VMEM((2 * N_PAD,), jnp.float32),
        pltpu.VMEM((2 * N_PAD,), jnp.float32),
    ],
)
def _sc_segsum_out(table_hbm, src_hbm, dst_hbm, zero_hbm, out_hbm,
                   isrc, idst, table_v, acc):
    c = lax.axis_index("c")
    s = lax.axis_index("s")
    pltpu.sync_copy(table_hbm, table_v)
    pltpu.sync_copy(zero_hbm, acc)
    pltpu.sync_copy(src_hbm.at[c, s], isrc)
    pltpu.sync_copy(dst_hbm.at[c, s], idst)

    def body(j, carry):
        for k in range(EC // 16):
            sb = isrc[j, pl.ds(k * 16, 16)] * 2
            db = idst[j, pl.ds(k * 16, 16)] * 2
            g0 = plsc.load_gather(table_v, [sb])
            g1 = plsc.load_gather(table_v, [sb + 1])
            plsc.addupdate_scatter(acc, [db], g0)
            plsc.addupdate_scatter(acc, [db + 1], g1)
        return carry

    lax.fori_loop(0, CH, body, 0)
    pltpu.sync_copy(acc, out_hbm.at[c, s])


def _dinv_block(deg_blk):
    deg = jnp.sum(deg_blk, axis=1, keepdims=True) + 1.0
    return lax.rsqrt(deg)


def _tc_hs(x_ref, w_ref, deg_ref, hs_ref):
    dinv = _dinv_block(deg_ref[...])
    hs_ref[...] = jnp.dot(x_ref[...], w_ref[...],
                          preferred_element_type=jnp.float32) * dinv


def _tc_layer2(accp_ref, hs_ref, deg_ref, w2_ref, b1_ref, gs_ref):
    dinv = _dinv_block(deg_ref[...])
    out1 = dinv * (accp_ref[0] + accp_ref[1] + hs_ref[...]) + b1_ref[...]
    r = jnp.maximum(out1, 0.0)
    gs_ref[...] = jnp.dot(r, w2_ref[...],
                          preferred_element_type=jnp.float32) * dinv


def _tc_softmax(acc2_ref, gs_ref, deg_ref, b2_ref, out_ref):
    dinv = _dinv_block(deg_ref[...])
    acc2 = jnp.sum(acc2_ref[...], axis=2)
    t = dinv * (acc2 + gs_ref[...]) + b2_ref[...]
    l0 = t[:, 0:1]
    l1 = t[:, 1:2]
    m = jnp.maximum(l0, l1)
    e0 = jnp.exp(l0 - m)
    e1 = jnp.exp(l1 - m)
    inv = 1.0 / (e0 + e1)
    out_ref[...] = jnp.concatenate([e0 * inv, e1 * inv], axis=1)


def kernel(x, edge_index, W1, b1, W2, b2):
    ei = edge_index.astype(jnp.int32)
    src = jnp.concatenate(
        [ei[0], jnp.zeros((E_PAD - ei.shape[1],), jnp.int32)])
    dst = jnp.concatenate(
        [ei[1], jnp.full((E_PAD - ei.shape[1],), N_NODES, jnp.int32)])
    src128 = src.reshape(NC, NS, CH, EC)
    dst128 = dst.reshape(NC, NS, CH, EC)

    x_pad = jnp.zeros((N_PAD, D_FEAT), jnp.float32).at[:N_NODES].set(x)
    b1r = b1.reshape(1, D_HID)
    b2r = b2.reshape(1, D_OUT)
    zero1 = jnp.zeros((N_PAD,), jnp.float32)
    zero2 = jnp.zeros((2 * N_PAD,), jnp.float32)
    zero128 = jnp.zeros((RPT, D_HID), jnp.float32)

    degp = _sc_degree(dst128, zero1)
    deg_t = degp.reshape(NW, N_PAD).transpose(1, 0)

    hs = pl.pallas_call(
        _tc_hs,
        grid=(GRID,),
        in_specs=[
            pl.BlockSpec((BR, D_FEAT), lambda i: (i, 0)),
            pl.BlockSpec((D_FEAT, D_HID), lambda i: (0, 0)),
            pl.BlockSpec((BR, NW), lambda i: (i, 0)),
        ],
        out_specs=pl.BlockSpec((BR, D_HID), lambda i: (i, 0)),
        out_shape=jax.ShapeDtypeStruct((N_PAD, D_HID), jnp.float32),
    )(x_pad, W1, deg_t)

    accp = _sc_segsum_hid(hs, src128, dst128, zero128)

    gs = pl.pallas_call(
        _tc_layer2,
        grid=(GRID,),
        in_specs=[
            pl.BlockSpec((NC, BR, D_HID), lambda i: (0, i, 0)),
            pl.BlockSpec((BR, D_HID), lambda i: (i, 0)),
            pl.BlockSpec((BR, NW), lambda i: (i, 0)),
            pl.BlockSpec((D_HID, D_OUT), lambda i: (0, 0)),
            pl.BlockSpec((1, D_HID), lambda i: (0, 0)),
        ],
        out_specs=pl.BlockSpec((BR, D_OUT), lambda i: (i, 0)),
        out_shape=jax.ShapeDtypeStruct((N_PAD, D_OUT), jnp.float32),
    )(accp, hs, deg_t, W2, b1r)

    acc2p = _sc_segsum_out(gs.reshape(2 * N_PAD), src128, dst128, zero2)
    acc2_t = acc2p.reshape(NW, N_PAD, 2).transpose(1, 2, 0)

    out = pl.pallas_call(
        _tc_softmax,
        grid=(GRID,),
        in_specs=[
            pl.BlockSpec((BR, D_OUT, NW), lambda i: (i, 0, 0)),
            pl.BlockSpec((BR, D_OUT), lambda i: (i, 0)),
            pl.BlockSpec((BR, NW), lambda i: (i, 0)),
            pl.BlockSpec((1, D_OUT), lambda i: (0, 0)),
        ],
        out_specs=pl.BlockSpec((BR, D_OUT), lambda i: (i, 0)),
        out_shape=jax.ShapeDtypeStruct((N_PAD, D_OUT), jnp.float32),
    )(acc2_t, gs, deg_t, b2r)

    return out[:N_NODES]
